# Initial kernel scaffold; baseline (speedup 1.0000x reference)
import functools

import jax
import jax.numpy as jnp
from jax import lax
from jax.experimental import pallas as pl
from jax.experimental.pallas import tpu as pltpu
from jax.experimental.pallas import tpu_sc as plsc

N = 10000
D = 128
LATENT = 64
MAX_LIG_NODES = 48

NC = 2
NS = 16
LANE = 16
CHUNK = 128
GP = CHUNK // LANE

NP = 10112
ROWS_TEC = NP // NS

NEG_BIG = -1e30


def _row_chunks(total, step):
    out = []
    off = 0
    while off < total:
        out.append((off, min(step, total - off)))
        off += step
    return out


def _edge_pass(src_r, dst_r, se_r, s_src, s_dst, h):
    R = src_r.shape[0]
    assert R % (NC * NS) == 0
    nch = R // NS
    nch2 = nch // NC

    mesh = plsc.VectorSubcoreMesh(core_axis_name="c", subcore_axis_name="s",
                                  num_cores=NC, num_subcores=NS)

    @functools.partial(
        pl.kernel,
        out_type=jax.ShapeDtypeStruct((NC, NP, D), jnp.float32),
        mesh=mesh,
        scratch_types=[
            pltpu.VMEM((NP,), jnp.float32),
            pltpu.VMEM((NP,), jnp.float32),
            pltpu.VMEM((NP,), jnp.float32),
            pltpu.VMEM((nch, CHUNK), jnp.int32),
            pltpu.VMEM((nch, CHUNK), jnp.int32),
            pltpu.VMEM((nch, CHUNK), jnp.float32),
            pltpu.VMEM((CHUNK, D), jnp.float32),
            pltpu.VMEM((CHUNK,), jnp.float32),
            pltpu.VMEM_SHARED((NP,), jnp.float32),
            pltpu.VMEM_SHARED((NP, D), jnp.float32),
            pltpu.SemaphoreType.DMA,
        ],
    )
    def edge_kernel(src_hbm, dst_hbm, se_hbm, ssrc_hbm, sdst_hbm, h_hbm,
                    out_hbm, ssrc_t, sdst_t, den_t, srcc, dstc, exc,
                    rowb, coefb, sp_den, sp_out, sem):
        c = lax.axis_index("c")
        s = lax.axis_index("s")

        pltpu.sync_copy(ssrc_hbm, ssrc_t)
        pltpu.sync_copy(sdst_hbm, sdst_t)
        row0 = s * nch
        pltpu.sync_copy(src_hbm.at[pl.ds(row0, nch)], srcc)
        pltpu.sync_copy(dst_hbm.at[pl.ds(row0, nch)], dstc)
        pltpu.sync_copy(se_hbm.at[pl.ds(row0, nch)], exc)

        z16 = jnp.zeros((LANE,), jnp.float32)

        def zrow(i, _):
            jr = i // GP
            kk = (i % GP) * LANE
            rowb[jr, pl.ds(kk, LANE)] = z16
            return 0
        lax.fori_loop(0, CHUNK * (D // LANE), zrow, 0)

        def zden(i, _):
            den_t[pl.ds(i * LANE, LANE)] = z16
            return 0
        lax.fori_loop(0, NP // LANE, zden, 0)

        r0 = s * ROWS_TEC
        pltpu.sync_copy(den_t.at[pl.ds(r0, ROWS_TEC)],
                        sp_den.at[pl.ds(r0, ROWS_TEC)])
        for off, ln in _row_chunks(ROWS_TEC, CHUNK):
            pltpu.sync_copy(rowb.at[pl.ds(0, ln)],
                            sp_out.at[pl.ds(r0 + off, ln)])

        def p1(t, _):
            j = t // GP
            kk = (t % GP) * LANE
            src16 = srcc[j, pl.ds(kk, LANE)]
            dst16 = dstc[j, pl.ds(kk, LANE)]
            a = (plsc.load_gather(ssrc_t, [src16])
                 + plsc.load_gather(sdst_t, [dst16])
                 + exc[j, pl.ds(kk, LANE)])
            a = jnp.where(a > 0.0, a, 0.2 * a)
            exc[j, pl.ds(kk, LANE)] = jnp.exp(a)
            return 0
        lax.fori_loop(0, nch * GP, p1, 0)

        plsc.subcore_barrier()

        def addb(j, _):
            pltpu.sync_copy(exc.at[j], sp_den.at[dstc.at[j]], add=True)
            return 0
        lax.fori_loop(0, nch, addb, 0)
        plsc.subcore_barrier()
        pltpu.sync_copy(sp_den, den_t)

        ch0 = c * nch2

        def p2(j, _):
            jj = ch0 + j
            pltpu.async_copy(h_hbm.at[srcc.at[jj]], rowb, sem).wait()

            def cf(k, _):
                kk = k * LANE
                dst16 = dstc[jj, pl.ds(kk, LANE)]
                den = plsc.load_gather(den_t, [dst16])
                coefb[pl.ds(kk, LANE)] = (
                    exc[jj, pl.ds(kk, LANE)] / (den + 1e-16))
                return 0
            lax.fori_loop(0, GP, cf, 0)

            def scale(i, _):
                cfi = coefb[i]
                for dd in range(D // LANE):
                    rowb[i, pl.ds(dd * LANE, LANE)] = (
                        rowb[i, pl.ds(dd * LANE, LANE)] * cfi)
                return 0
            lax.fori_loop(0, CHUNK, scale, 0)

            pltpu.sync_copy(rowb, sp_out.at[dstc.at[jj]], add=True)
            return 0
        lax.fori_loop(0, nch2, p2, 0)
        plsc.subcore_barrier()

        for off, ln in _row_chunks(ROWS_TEC, CHUNK):
            pltpu.sync_copy(sp_out.at[pl.ds(r0 + off, ln)],
                            rowb.at[pl.ds(0, ln)])
            pltpu.sync_copy(rowb.at[pl.ds(0, ln)],
                            out_hbm.at[c, pl.ds(r0 + off, ln)])

    return edge_kernel(src_r, dst_r, se_r, s_src, s_dst, h)


def _dense_node(x, W, A):
    def body(x_ref, w_ref, a_ref, h_ref, s_ref):
        h = jnp.dot(x_ref[...], w_ref[...], preferred_element_type=jnp.float32)
        h_ref[...] = h
        s_ref[...] = jnp.dot(h, a_ref[...], preferred_element_type=jnp.float32)
    return pl.pallas_call(
        body,
        out_shape=[jax.ShapeDtypeStruct((NP, D), jnp.float32),
                   jax.ShapeDtypeStruct((NP, 2), jnp.float32)],
    )(x, W, A)


def _edge_scores(ea_r, B):
    R8 = ea_r.shape[0]
    blk = 4000
    assert R8 % blk == 0
    def body(ea_ref, b_ref, o_ref):
        o_ref[...] = jnp.dot(ea_ref[...], b_ref[...],
                             preferred_element_type=jnp.float32)
    return pl.pallas_call(
        body,
        grid=(R8 // blk,),
        in_specs=[pl.BlockSpec((blk, 128), lambda i: (i, 0)),
                  pl.BlockSpec((128, 16), lambda i: (0, 0))],
        out_specs=pl.BlockSpec((blk, 16), lambda i: (i, 0)),
        out_shape=jax.ShapeDtypeStruct((R8, 16), jnp.float32),
    )(ea_r, B)


def _bn_relu_dense(o0, o1, b, gamma, beta, W2, A2):
    def body(o0_ref, o1_ref, b_ref, g_ref, be_ref, w_ref, a_ref,
             h_ref, s_ref):
        t = o0_ref[...] + o1_ref[...] + b_ref[...]
        rmask = lax.broadcasted_iota(jnp.int32, (NP, 1), 0) < N
        tm = jnp.where(rmask, t, 0.0)
        m = jnp.sum(tm, axis=0, keepdims=True) / N
        ex2 = jnp.sum(tm * tm, axis=0, keepdims=True) / N
        var = ex2 - m * m
        y = (t - m) / jnp.sqrt(var + 1e-5) * g_ref[...] + be_ref[...]
        y = jnp.maximum(y, 0.0)
        y = jnp.where(rmask, y, 0.0)
        h = jnp.dot(y, w_ref[...], preferred_element_type=jnp.float32)
        h_ref[...] = h
        s_ref[...] = jnp.dot(h, a_ref[...], preferred_element_type=jnp.float32)
    return pl.pallas_call(
        body,
        out_shape=[jax.ShapeDtypeStruct((NP, D), jnp.float32),
                   jax.ShapeDtypeStruct((NP, 2), jnp.float32)],
    )(o0, o1, b.reshape(1, D), gamma.reshape(1, D), beta.reshape(1, D),
      W2, A2)


def _head(po0, po1, pb, pg, pbe, lo0, lo1, lb, lg, lbe,
          eps, W_mu, b_mu, W_lv, b_lv, W_dec, b_dec):
    def body(po0_ref, po1_ref, pb_ref, pg_ref, pbe_ref,
             lo0_ref, lo1_ref, lb_ref, lg_ref, lbe_ref,
             eps_ref, wmu_ref, bmu_ref, wlv_ref, blv_ref,
             wdec_ref, bdec_ref, nf_ref, mu_ref, lv_ref):
        rmask = lax.broadcasted_iota(jnp.int32, (NP, 1), 0) < N

        def pool(t, g_, be_):
            tm = jnp.where(rmask, t, 0.0)
            m = jnp.sum(tm, axis=0, keepdims=True) / N
            ex2 = jnp.sum(tm * tm, axis=0, keepdims=True) / N
            var = ex2 - m * m
            y = (t - m) / jnp.sqrt(var + 1e-5) * g_ + be_
            y = jnp.maximum(y, 0.0)
            y = jnp.where(rmask, y, 0.0)
            return jnp.sum(y, axis=0, keepdims=True) / N

        gp = pool(po0_ref[...] + po1_ref[...] + pb_ref[...],
                  pg_ref[...], pbe_ref[...])
        gl = pool(lo0_ref[...] + lo1_ref[...] + lb_ref[...],
                  lg_ref[...], lbe_ref[...])
        hcat = jnp.concatenate([gp, gl], axis=1)
        mu = jnp.dot(hcat, wmu_ref[...],
                     preferred_element_type=jnp.float32) + bmu_ref[...]
        lv = jnp.dot(hcat, wlv_ref[...],
                     preferred_element_type=jnp.float32) + blv_ref[...]
        z = mu + jnp.exp(0.5 * lv) * eps_ref[...]
        cond = jnp.concatenate([z, gp], axis=1)
        nf = jnp.dot(cond, wdec_ref[...],
                     preferred_element_type=jnp.float32) + bdec_ref[...]
        nf_ref[...] = jnp.broadcast_to(nf, (MAX_LIG_NODES, D))
        mu_ref[...] = mu
        lv_ref[...] = lv
    return pl.pallas_call(
        body,
        out_shape=[jax.ShapeDtypeStruct((MAX_LIG_NODES, D), jnp.float32),
                   jax.ShapeDtypeStruct((1, LATENT), jnp.float32),
                   jax.ShapeDtypeStruct((1, LATENT), jnp.float32)],
    )(po0, po1, pb.reshape(1, D), pg.reshape(1, D), pbe.reshape(1, D),
      lo0, lo1, lb.reshape(1, D), lg.reshape(1, D), lbe.reshape(1, D),
      eps.reshape(1, LATENT), W_mu, b_mu.reshape(1, LATENT),
      W_lv, b_lv.reshape(1, LATENT), W_dec, b_dec.reshape(1, D))


def _fold_edge_selector(p):
    v = p["We"] @ p["a_e"]
    idx = jnp.arange(128)
    vt = jnp.tile(v, 8)
    return jnp.where((idx // 16)[:, None] == jnp.arange(8)[None, :],
                     vt[:, None], 0.0).astype(jnp.float32)


def _attn_vec(p):
    return jnp.stack([p["a_src"], p["a_dst"]], axis=1)


def _pad_edges(ei, E):
    group = NC * NS * CHUNK
    Epad = -(-E // group) * group
    src = jnp.pad(ei[0].astype(jnp.int32), (0, Epad - E))
    dst = jnp.pad(ei[1].astype(jnp.int32), (0, Epad - E))
    return (src.reshape(Epad // CHUNK, CHUNK),
            dst.reshape(Epad // CHUNK, CHUNK), Epad)


def _pad_se(se_flat, Epad):
    E = se_flat.shape[0]
    se = jnp.pad(se_flat, (0, Epad - E), constant_values=NEG_BIG)
    return se.reshape(Epad // CHUNK, CHUNK)


def kernel(prot_x, prot_edge_index, prot_edge_attr,
           lig_x, lig_edge_index, lig_edge_attr, eps, params):
    P = params
    px = jnp.pad(prot_x, ((0, NP - N), (0, 0)))
    lx = jnp.pad(lig_x, ((0, NP - N), (0, 0)))
    psrc, pdst, pEpad = _pad_edges(prot_edge_index, prot_edge_attr.shape[0])
    lsrc, ldst, lEpad = _pad_edges(lig_edge_index, lig_edge_attr.shape[0])
    eap = prot_edge_attr.reshape(-1, 128)
    eal = lig_edge_attr.reshape(-1, 128)

    Bp = jnp.concatenate([_fold_edge_selector(P["p_conv1"]),
                          _fold_edge_selector(P["p_conv2"])], axis=1)
    Bl = jnp.concatenate([_fold_edge_selector(P["l_conv1"]),
                          _fold_edge_selector(P["l_conv2"])], axis=1)

    sep = _edge_scores(eap, Bp)
    sel = _edge_scores(eal, Bl)
    sep1 = _pad_se(sep[:, :8].reshape(-1), pEpad)
    sep2 = _pad_se(sep[:, 8:].reshape(-1), pEpad)
    sel1 = _pad_se(sel[:, :8].reshape(-1), lEpad)
    sel2 = _pad_se(sel[:, 8:].reshape(-1), lEpad)

    hp1, Sp1 = _dense_node(px, P["p_conv1"]["W"], _attn_vec(P["p_conv1"]))
    hl1, Sl1 = _dense_node(lx, P["l_conv1"]["W"], _attn_vec(P["l_conv1"]))
    op1 = _edge_pass(psrc, pdst, sep1, Sp1[:, 0], Sp1[:, 1], hp1)
    ol1 = _edge_pass(lsrc, ldst, sel1, Sl1[:, 0], Sl1[:, 1], hl1)

    hp2, Sp2 = _bn_relu_dense(op1[0], op1[1], P["p_conv1"]["b"],
                              P["p_bn1_g"], P["p_bn1_b"],
                              P["p_conv2"]["W"], _attn_vec(P["p_conv2"]))
    hl2, Sl2 = _bn_relu_dense(ol1[0], ol1[1], P["l_conv1"]["b"],
                              P["l_bn1_g"], P["l_bn1_b"],
                              P["l_conv2"]["W"], _attn_vec(P["l_conv2"]))

    op2 = _edge_pass(psrc, pdst, sep2, Sp2[:, 0], Sp2[:, 1], hp2)
    ol2 = _edge_pass(lsrc, ldst, sel2, Sl2[:, 0], Sl2[:, 1], hl2)

    nf, mu, lv = _head(op2[0], op2[1], P["p_conv2"]["b"],
                       P["p_bn2_g"], P["p_bn2_b"],
                       ol2[0], ol2[1], P["l_conv2"]["b"],
                       P["l_bn2_g"], P["l_bn2_b"],
                       eps, P["W_mu"], P["b_mu"], P["W_lv"], P["b_lv"],
                       P["W_dec"], P["b_dec"])
    return (nf[None], mu.reshape(LATENT), lv.reshape(LATENT))


# speedup vs baseline: 10.8593x; 10.8593x over previous
import functools

import jax
import jax.numpy as jnp
from jax import lax
from jax.experimental import pallas as pl
from jax.experimental.pallas import tpu as pltpu
from jax.experimental.pallas import tpu_sc as plsc

N = 10000
D = 128
LATENT = 64
MAX_LIG_NODES = 48

NC = 2
NS = 16
LANE = 16
CHUNK = 128
GP = CHUNK // LANE
BR = 4

NP = 10112
ROWS_TEC = NP // NS

NEG_BIG = -1e30


def _row_chunks(total, step):
    out = []
    off = 0
    while off < total:
        out.append((off, min(step, total - off)))
        off += step
    return out


def _edge_pass(src_r, dst_r, se_r, s_src, s_dst, h):
    R = src_r.shape[0]
    nch = R // NS
    nch2 = nch // NC
    nblk2 = nch2 // BR
    assert R % (NS * NC * BR) == 0

    mesh = plsc.VectorSubcoreMesh(core_axis_name="c", subcore_axis_name="s",
                                  num_cores=NC, num_subcores=NS)

    @functools.partial(
        pl.kernel,
        out_type=jax.ShapeDtypeStruct((NC, NP, D), jnp.float32),
        mesh=mesh,
        compiler_params=pltpu.CompilerParams(needs_layout_passes=False),
        scratch_types=[
            pltpu.VMEM((NP,), jnp.float32),
            pltpu.VMEM((NP,), jnp.float32),
            pltpu.VMEM((BR, CHUNK), jnp.int32),
            pltpu.VMEM((BR, CHUNK), jnp.int32),
            pltpu.VMEM((BR, CHUNK), jnp.float32),
            pltpu.VMEM((nch2, CHUNK), jnp.float32),
            pltpu.VMEM((CHUNK, D), jnp.float32),
            pltpu.VMEM((CHUNK,), jnp.float32),
            pltpu.VMEM_SHARED((NP,), jnp.float32),
            pltpu.VMEM_SHARED((NP, D), jnp.float32),
            pltpu.SemaphoreType.DMA,
        ],
    )
    def edge_kernel(src_hbm, dst_hbm, se_hbm, ssrc_hbm, sdst_hbm, h_hbm,
                    out_hbm, ssrc_t, sdst_t, src_b, dst_b, se_b,
                    exc, rowb, denb, sp_den, sp_out, sem):
        c = lax.axis_index("c")
        s = lax.axis_index("s")

        pltpu.sync_copy(ssrc_hbm, ssrc_t)
        pltpu.sync_copy(sdst_hbm, sdst_t)
        row0 = s * nch

        z16 = jnp.zeros((LANE,), jnp.float32)

        def zrow(i, _):
            jr = i // GP
            kk = (i % GP) * LANE
            rowb[jr, pl.ds(kk, LANE)] = z16
            return 0
        lax.fori_loop(0, CHUNK * (D // LANE), zrow, 0)

        def zden(i, _):
            denb[pl.ds(i * LANE, LANE)] = z16
            return 0
        lax.fori_loop(0, CHUNK // LANE, zden, 0)

        r0 = s * ROWS_TEC
        for off, ln in _row_chunks(ROWS_TEC, CHUNK):
            pltpu.sync_copy(denb.at[pl.ds(0, ln)],
                            sp_den.at[pl.ds(r0 + off, ln)])
            pltpu.sync_copy(rowb.at[pl.ds(0, ln)],
                            sp_out.at[pl.ds(r0 + off, ln)])

        def p1_block(gb, store, b2):
            blk0 = row0 + gb * BR
            pltpu.sync_copy(src_hbm.at[pl.ds(blk0, BR)], src_b)
            pltpu.sync_copy(dst_hbm.at[pl.ds(blk0, BR)], dst_b)
            pltpu.sync_copy(se_hbm.at[pl.ds(blk0, BR)], se_b)

            def p1(t, _):
                j = t // GP
                kk = (t % GP) * LANE
                src16 = src_b[j, pl.ds(kk, LANE)]
                dst16 = dst_b[j, pl.ds(kk, LANE)]
                a = (plsc.load_gather(ssrc_t, [src16])
                     + plsc.load_gather(sdst_t, [dst16])
                     + se_b[j, pl.ds(kk, LANE)])
                a = jnp.where(a > 0.0, a, 0.2 * a)
                ex = jnp.exp(a)
                se_b[j, pl.ds(kk, LANE)] = ex
                if store:
                    exc[b2 * BR + j, pl.ds(kk, LANE)] = ex
                return 0
            lax.fori_loop(0, BR * GP, p1, 0)

            def addb(j, _):
                pltpu.sync_copy(se_b.at[j], sp_den.at[dst_b.at[j]],
                                add=True)
                return 0
            lax.fori_loop(0, BR, addb, 0)
            return 0

        plsc.subcore_barrier()
        lax.fori_loop(0, nblk2,
                      lambda b2, _: p1_block(c * nblk2 + b2, True, b2), 0)
        lax.fori_loop(0, nblk2,
                      lambda b2, _: p1_block((1 - c) * nblk2 + b2, False, b2),
                      0)
        plsc.subcore_barrier()

        def p2_block(b2, _):
            blk0 = row0 + (c * nblk2 + b2) * BR
            pltpu.sync_copy(src_hbm.at[pl.ds(blk0, BR)], src_b)
            pltpu.sync_copy(dst_hbm.at[pl.ds(blk0, BR)], dst_b)

            def p2(j, _):
                pltpu.async_copy(h_hbm.at[src_b.at[j]], rowb, sem).wait()
                jj = b2 * BR + j

                def scale(i, _):
                    cfi = plsc.load_gather(
                        exc, [jnp.full((LANE,), jj, jnp.int32),
                              jnp.full((LANE,), i, jnp.int32)])
                    for dd in range(D // LANE):
                        rowb[i, pl.ds(dd * LANE, LANE)] = (
                            rowb[i, pl.ds(dd * LANE, LANE)] * cfi)
                    return 0
                lax.fori_loop(0, CHUNK, scale, 0)

                pltpu.sync_copy(rowb, sp_out.at[dst_b.at[j]], add=True)
                return 0
            lax.fori_loop(0, BR, p2, 0)
            return 0
        lax.fori_loop(0, nblk2, p2_block, 0)
        plsc.subcore_barrier()

        for off, ln in _row_chunks(ROWS_TEC, CHUNK):
            pltpu.sync_copy(sp_den.at[pl.ds(r0 + off, ln)],
                            denb.at[pl.ds(0, ln)])
            pltpu.sync_copy(sp_out.at[pl.ds(r0 + off, ln)],
                            rowb.at[pl.ds(0, ln)])

            def norm(i, _):
                di = plsc.load_gather(denb, [jnp.full((LANE,), i, jnp.int32)])
                inv = 1.0 / (di + 1e-16)
                for dd in range(D // LANE):
                    rowb[i, pl.ds(dd * LANE, LANE)] = (
                        rowb[i, pl.ds(dd * LANE, LANE)] * inv)
                return 0
            lax.fori_loop(0, ln, norm, 0)
            pltpu.sync_copy(rowb.at[pl.ds(0, ln)],
                            out_hbm.at[c, pl.ds(r0 + off, ln)])

    return edge_kernel(src_r, dst_r, se_r, s_src, s_dst, h)


def _dense_node(x, W, A):
    def body(x_ref, w_ref, a_ref, h_ref, s_ref):
        h = jnp.dot(x_ref[...], w_ref[...], preferred_element_type=jnp.float32)
        h_ref[...] = h
        s_ref[...] = jnp.dot(h, a_ref[...], preferred_element_type=jnp.float32)
    return pl.pallas_call(
        body,
        out_shape=[jax.ShapeDtypeStruct((NP, D), jnp.float32),
                   jax.ShapeDtypeStruct((NP, 2), jnp.float32)],
    )(x, W, A)


def _edge_scores(ea_r, B):
    R8 = ea_r.shape[0]
    blk = 4000
    assert R8 % blk == 0
    def body(ea_ref, b_ref, o_ref):
        o_ref[...] = jnp.dot(ea_ref[...], b_ref[...],
                             preferred_element_type=jnp.float32)
    return pl.pallas_call(
        body,
        grid=(R8 // blk,),
        in_specs=[pl.BlockSpec((blk, 128), lambda i: (i, 0)),
                  pl.BlockSpec((128, 16), lambda i: (0, 0))],
        out_specs=pl.BlockSpec((blk, 16), lambda i: (i, 0)),
        out_shape=jax.ShapeDtypeStruct((R8, 16), jnp.float32),
    )(ea_r, B)


def _bn_relu(o, b_ref, g_ref, be_ref):
    t = o[0] + o[1] + b_ref
    rmask = lax.broadcasted_iota(jnp.int32, (NP, 1), 0) < N
    tm = jnp.where(rmask, t, 0.0)
    m = jnp.sum(tm, axis=0, keepdims=True) / N
    ex2 = jnp.sum(tm * tm, axis=0, keepdims=True) / N
    var = ex2 - m * m
    y = (t - m) / jnp.sqrt(var + 1e-5) * g_ref + be_ref
    y = jnp.maximum(y, 0.0)
    return jnp.where(rmask, y, 0.0)


def _bn_relu_dense(o, b, gamma, beta, W2, A2):
    def body(o_ref, b_ref, g_ref, be_ref, w_ref, a_ref, h_ref, s_ref):
        y = _bn_relu(o_ref[...], b_ref[...], g_ref[...], be_ref[...])
        h = jnp.dot(y, w_ref[...], preferred_element_type=jnp.float32)
        h_ref[...] = h
        s_ref[...] = jnp.dot(h, a_ref[...], preferred_element_type=jnp.float32)
    return pl.pallas_call(
        body,
        out_shape=[jax.ShapeDtypeStruct((NP, D), jnp.float32),
                   jax.ShapeDtypeStruct((NP, 2), jnp.float32)],
    )(o, b.reshape(1, D), gamma.reshape(1, D), beta.reshape(1, D), W2, A2)


def _head(po, pb, pg, pbe, lo, lb, lg, lbe,
          eps, W_mu, b_mu, W_lv, b_lv, W_dec, b_dec):
    def body(po_ref, pb_ref, pg_ref, pbe_ref,
             lo_ref, lb_ref, lg_ref, lbe_ref,
             eps_ref, wmu_ref, bmu_ref, wlv_ref, blv_ref,
             wdec_ref, bdec_ref, nf_ref, mu_ref, lv_ref):
        def pool(o, b_, g_, be_):
            y = _bn_relu(o, b_, g_, be_)
            return jnp.sum(y, axis=0, keepdims=True) / N

        gp = pool(po_ref[...], pb_ref[...], pg_ref[...], pbe_ref[...])
        gl = pool(lo_ref[...], lb_ref[...], lg_ref[...], lbe_ref[...])
        hcat = jnp.concatenate([gp, gl], axis=1)
        mu = jnp.dot(hcat, wmu_ref[...],
                     preferred_element_type=jnp.float32) + bmu_ref[...]
        lv = jnp.dot(hcat, wlv_ref[...],
                     preferred_element_type=jnp.float32) + blv_ref[...]
        z = mu + jnp.exp(0.5 * lv) * eps_ref[...]
        cond = jnp.concatenate([z, gp], axis=1)
        nf = jnp.dot(cond, wdec_ref[...],
                     preferred_element_type=jnp.float32) + bdec_ref[...]
        nf_ref[...] = jnp.broadcast_to(nf, (MAX_LIG_NODES, D))
        mu_ref[...] = mu
        lv_ref[...] = lv
    return pl.pallas_call(
        body,
        out_shape=[jax.ShapeDtypeStruct((MAX_LIG_NODES, D), jnp.float32),
                   jax.ShapeDtypeStruct((1, LATENT), jnp.float32),
                   jax.ShapeDtypeStruct((1, LATENT), jnp.float32)],
    )(po, pb.reshape(1, D), pg.reshape(1, D), pbe.reshape(1, D),
      lo, lb.reshape(1, D), lg.reshape(1, D), lbe.reshape(1, D),
      eps.reshape(1, LATENT), W_mu, b_mu.reshape(1, LATENT),
      W_lv, b_lv.reshape(1, LATENT), W_dec, b_dec.reshape(1, D))


def _fold_edge_selector(p):
    v = p["We"] @ p["a_e"]
    idx = jnp.arange(128)
    vt = jnp.tile(v, 8)
    return jnp.where((idx // 16)[:, None] == jnp.arange(8)[None, :],
                     vt[:, None], 0.0).astype(jnp.float32)


def _attn_vec(p):
    return jnp.stack([p["a_src"], p["a_dst"]], axis=1)


def _pad_edges(ei, E):
    group = NS * NC * BR * CHUNK
    Epad = -(-E // group) * group
    src = jnp.pad(ei[0].astype(jnp.int32), (0, Epad - E))
    dst = jnp.pad(ei[1].astype(jnp.int32), (0, Epad - E))
    return (src.reshape(Epad // CHUNK, CHUNK),
            dst.reshape(Epad // CHUNK, CHUNK), Epad)


def _pad_se(se_flat, Epad):
    E = se_flat.shape[0]
    se = jnp.pad(se_flat, (0, Epad - E), constant_values=NEG_BIG)
    return se.reshape(Epad // CHUNK, CHUNK)


def kernel(prot_x, prot_edge_index, prot_edge_attr,
           lig_x, lig_edge_index, lig_edge_attr, eps, params):
    P = params
    px = jnp.pad(prot_x, ((0, NP - N), (0, 0)))
    lx = jnp.pad(lig_x, ((0, NP - N), (0, 0)))
    psrc, pdst, pEpad = _pad_edges(prot_edge_index, prot_edge_attr.shape[0])
    lsrc, ldst, lEpad = _pad_edges(lig_edge_index, lig_edge_attr.shape[0])
    eap = prot_edge_attr.reshape(-1, 128)
    eal = lig_edge_attr.reshape(-1, 128)

    Bp = jnp.concatenate([_fold_edge_selector(P["p_conv1"]),
                          _fold_edge_selector(P["p_conv2"])], axis=1)
    Bl = jnp.concatenate([_fold_edge_selector(P["l_conv1"]),
                          _fold_edge_selector(P["l_conv2"])], axis=1)

    sep = _edge_scores(eap, Bp)
    sel = _edge_scores(eal, Bl)
    sep1 = _pad_se(sep[:, :8].reshape(-1), pEpad)
    sep2 = _pad_se(sep[:, 8:].reshape(-1), pEpad)
    sel1 = _pad_se(sel[:, :8].reshape(-1), lEpad)
    sel2 = _pad_se(sel[:, 8:].reshape(-1), lEpad)

    hp1, Sp1 = _dense_node(px, P["p_conv1"]["W"], _attn_vec(P["p_conv1"]))
    hl1, Sl1 = _dense_node(lx, P["l_conv1"]["W"], _attn_vec(P["l_conv1"]))
    op1 = _edge_pass(psrc, pdst, sep1, Sp1[:, 0], Sp1[:, 1], hp1)
    ol1 = _edge_pass(lsrc, ldst, sel1, Sl1[:, 0], Sl1[:, 1], hl1)

    hp2, Sp2 = _bn_relu_dense(op1, P["p_conv1"]["b"],
                              P["p_bn1_g"], P["p_bn1_b"],
                              P["p_conv2"]["W"], _attn_vec(P["p_conv2"]))
    hl2, Sl2 = _bn_relu_dense(ol1, P["l_conv1"]["b"],
                              P["l_bn1_g"], P["l_bn1_b"],
                              P["l_conv2"]["W"], _attn_vec(P["l_conv2"]))

    op2 = _edge_pass(psrc, pdst, sep2, Sp2[:, 0], Sp2[:, 1], hp2)
    ol2 = _edge_pass(lsrc, ldst, sel2, Sl2[:, 0], Sl2[:, 1], hl2)

    nf, mu, lv = _head(op2, P["p_conv2"]["b"], P["p_bn2_g"], P["p_bn2_b"],
                       ol2, P["l_conv2"]["b"], P["l_bn2_g"], P["l_bn2_b"],
                       eps, P["W_mu"], P["b_mu"], P["W_lv"], P["b_lv"],
                       P["W_dec"], P["b_dec"])
    return (nf[None], mu.reshape(LATENT), lv.reshape(LATENT))


# speedup vs baseline: 23.5726x; 2.1707x over previous
import functools

import jax
import jax.numpy as jnp
from jax import lax
from jax.experimental import pallas as pl
from jax.experimental.pallas import tpu as pltpu
from jax.experimental.pallas import tpu_sc as plsc

N = 10000
D = 128
LATENT = 64
MAX_LIG_NODES = 48

NC = 2
NS = 16
LANE = 16
CHUNK = 128
GP = CHUNK // LANE
BR = 8

NP = 10112
ROWS_TEC = NP // NS

NEG_BIG = -1e30


def _row_chunks(total, step):
    out = []
    off = 0
    while off < total:
        out.append((off, min(step, total - off)))
        off += step
    return out


def _edge_pass(src_r, dst_r, se_r, s_src, s_dst, h, n_edges):
    R = src_r.shape[0]
    nch2 = R // (NS * NC)
    nblk2 = nch2 // BR
    assert R % (NS * NC * BR) == 0
    RR = -(-n_edges // CHUNK)

    mesh = plsc.VectorSubcoreMesh(core_axis_name="c", subcore_axis_name="s",
                                  num_cores=NC, num_subcores=NS)

    @functools.partial(
        pl.kernel,
        out_type=(jax.ShapeDtypeStruct((NC, NP, D), jnp.float32),
                  jax.ShapeDtypeStruct((NC * NP,), jnp.float32)),
        mesh=mesh,
        compiler_params=pltpu.CompilerParams(needs_layout_passes=False),
        scratch_types=[
            pltpu.VMEM((NP,), jnp.float32),
            pltpu.VMEM((NP,), jnp.float32),
            pltpu.VMEM((BR, CHUNK), jnp.int32),
            pltpu.VMEM((BR, CHUNK), jnp.int32),
            pltpu.VMEM((BR, CHUNK), jnp.float32),
            pltpu.VMEM((CHUNK, D), jnp.float32),
            pltpu.VMEM((CHUNK,), jnp.float32),
            pltpu.VMEM_SHARED((NP,), jnp.float32),
            pltpu.VMEM_SHARED((NP, D), jnp.float32),
            pltpu.SemaphoreType.DMA,
        ],
    )
    def edge_kernel(src_hbm, dst_hbm, se_hbm, ssrc_hbm, sdst_hbm, h_hbm,
                    out_hbm, den_hbm, ssrc_t, sdst_t, src_b, dst_b, se_b,
                    rowb, denb, sp_den, sp_out, sem):
        c = lax.axis_index("c")
        s = lax.axis_index("s")

        pltpu.sync_copy(ssrc_hbm, ssrc_t)
        pltpu.sync_copy(sdst_hbm, sdst_t)
        base = (c * NS + s) * nch2

        z16 = jnp.zeros((LANE,), jnp.float32)

        def zrow(i, _):
            jr = i // GP
            kk = (i % GP) * LANE
            rowb[jr, pl.ds(kk, LANE)] = z16
            return 0
        lax.fori_loop(0, CHUNK * (D // LANE), zrow, 0)

        def zden(i, _):
            denb[pl.ds(i * LANE, LANE)] = z16
            return 0
        lax.fori_loop(0, CHUNK // LANE, zden, 0)

        r0 = s * ROWS_TEC
        for off, ln in _row_chunks(ROWS_TEC, CHUNK):
            pltpu.sync_copy(denb.at[pl.ds(0, ln)],
                            sp_den.at[pl.ds(r0 + off, ln)])
            pltpu.sync_copy(rowb.at[pl.ds(0, ln)],
                            sp_out.at[pl.ds(r0 + off, ln)])

        plsc.subcore_barrier()

        def sweep_block(b, _):
            blk0 = base + b * BR

            @pl.when(blk0 < RR)
            def _():
                pltpu.sync_copy(src_hbm.at[pl.ds(blk0, BR)], src_b)
                pltpu.sync_copy(dst_hbm.at[pl.ds(blk0, BR)], dst_b)
                pltpu.sync_copy(se_hbm.at[pl.ds(blk0, BR)], se_b)

                def score(t, _):
                    j = t // GP
                    kk = (t % GP) * LANE
                    src16 = src_b[j, pl.ds(kk, LANE)]
                    dst16 = dst_b[j, pl.ds(kk, LANE)]
                    a = (plsc.load_gather(ssrc_t, [src16])
                         + plsc.load_gather(sdst_t, [dst16])
                         + se_b[j, pl.ds(kk, LANE)])
                    a = jnp.where(a > 0.0, a, 0.2 * a)
                    se_b[j, pl.ds(kk, LANE)] = jnp.exp(a)
                    return 0
                lax.fori_loop(0, BR * GP, score, 0)

                def row(j, _):
                    @pl.when(blk0 + j < RR)
                    def _():
                        pltpu.sync_copy(se_b.at[j], sp_den.at[dst_b.at[j]],
                                        add=True)
                        pltpu.async_copy(h_hbm.at[src_b.at[j]], rowb,
                                         sem).wait()

                        def scale(i, _):
                            cfi = plsc.load_gather(
                                se_b, [jnp.full((LANE,), j, jnp.int32),
                                       jnp.full((LANE,), i, jnp.int32)])
                            for dd in range(D // LANE):
                                rowb[i, pl.ds(dd * LANE, LANE)] = (
                                    rowb[i, pl.ds(dd * LANE, LANE)] * cfi)
                            return 0
                        lax.fori_loop(0, CHUNK, scale, 0)

                        pltpu.sync_copy(rowb, sp_out.at[dst_b.at[j]],
                                        add=True)
                    return 0
                lax.fori_loop(0, BR, row, 0)
            return 0
        lax.fori_loop(0, nblk2, sweep_block, 0)
        plsc.subcore_barrier()

        for off, ln in _row_chunks(ROWS_TEC, CHUNK):
            pltpu.sync_copy(sp_out.at[pl.ds(r0 + off, ln)],
                            rowb.at[pl.ds(0, ln)])
            pltpu.sync_copy(rowb.at[pl.ds(0, ln)],
                            out_hbm.at[c, pl.ds(r0 + off, ln)])
        for k in range(5):
            cid = s * 5 + k

            @pl.when(cid < NP // CHUNK)
            def _():
                pltpu.sync_copy(sp_den.at[pl.ds(cid * CHUNK, CHUNK)], denb)
                pltpu.sync_copy(
                    denb, den_hbm.at[pl.ds(c * NP + cid * CHUNK, CHUNK)])

    return edge_kernel(src_r, dst_r, se_r, s_src, s_dst, h)


def _dense_node(x, W, A):
    def body(x_ref, w_ref, a_ref, h_ref, s_ref):
        h = jnp.dot(x_ref[...], w_ref[...], preferred_element_type=jnp.float32)
        h_ref[...] = h
        s_ref[...] = jnp.dot(h, a_ref[...], preferred_element_type=jnp.float32)
    return pl.pallas_call(
        body,
        out_shape=[jax.ShapeDtypeStruct((NP, D), jnp.float32),
                   jax.ShapeDtypeStruct((NP, 2), jnp.float32)],
    )(x, W, A)


def _edge_scores(ea_r, B):
    R8 = ea_r.shape[0]
    blk = 4000
    assert R8 % blk == 0
    def body(ea_ref, b_ref, o_ref):
        o_ref[...] = jnp.dot(ea_ref[...], b_ref[...],
                             preferred_element_type=jnp.float32)
    return pl.pallas_call(
        body,
        grid=(R8 // blk,),
        in_specs=[pl.BlockSpec((blk, 128), lambda i: (i, 0)),
                  pl.BlockSpec((128, 16), lambda i: (0, 0))],
        out_specs=pl.BlockSpec((blk, 16), lambda i: (i, 0)),
        out_shape=jax.ShapeDtypeStruct((R8, 16), jnp.float32),
    )(ea_r, B)


def _bn_relu(o, den, b_ref, g_ref, be_ref):
    t = (o[0] + o[1]) / (den[0] + den[1] + 1e-16) + b_ref
    rmask = lax.broadcasted_iota(jnp.int32, (NP, 1), 0) < N
    tm = jnp.where(rmask, t, 0.0)
    m = jnp.sum(tm, axis=0, keepdims=True) / N
    ex2 = jnp.sum(tm * tm, axis=0, keepdims=True) / N
    var = ex2 - m * m
    y = (t - m) / jnp.sqrt(var + 1e-5) * g_ref + be_ref
    y = jnp.maximum(y, 0.0)
    return jnp.where(rmask, y, 0.0)


def _bn_relu_dense(o, den, b, gamma, beta, W2, A2):
    def body(o_ref, d_ref, b_ref, g_ref, be_ref, w_ref, a_ref, h_ref, s_ref):
        y = _bn_relu(o_ref[...], d_ref[...], b_ref[...], g_ref[...],
                     be_ref[...])
        h = jnp.dot(y, w_ref[...], preferred_element_type=jnp.float32)
        h_ref[...] = h
        s_ref[...] = jnp.dot(h, a_ref[...], preferred_element_type=jnp.float32)
    return pl.pallas_call(
        body,
        out_shape=[jax.ShapeDtypeStruct((NP, D), jnp.float32),
                   jax.ShapeDtypeStruct((NP, 2), jnp.float32)],
    )(o, den, b.reshape(1, D), gamma.reshape(1, D), beta.reshape(1, D),
      W2, A2)


def _head(po, pd, pb, pg, pbe, lo, ld, lb, lg, lbe,
          eps, W_mu, b_mu, W_lv, b_lv, W_dec, b_dec):
    def body(po_ref, pd_ref, pb_ref, pg_ref, pbe_ref,
             lo_ref, ld_ref, lb_ref, lg_ref, lbe_ref,
             eps_ref, wmu_ref, bmu_ref, wlv_ref, blv_ref,
             wdec_ref, bdec_ref, nf_ref, mu_ref, lv_ref):
        def pool(o, d_, b_, g_, be_):
            y = _bn_relu(o, d_, b_, g_, be_)
            return jnp.sum(y, axis=0, keepdims=True) / N

        gp = pool(po_ref[...], pd_ref[...], pb_ref[...], pg_ref[...],
                  pbe_ref[...])
        gl = pool(lo_ref[...], ld_ref[...], lb_ref[...], lg_ref[...],
                  lbe_ref[...])
        hcat = jnp.concatenate([gp, gl], axis=1)
        mu = jnp.dot(hcat, wmu_ref[...],
                     preferred_element_type=jnp.float32) + bmu_ref[...]
        lv = jnp.dot(hcat, wlv_ref[...],
                     preferred_element_type=jnp.float32) + blv_ref[...]
        z = mu + jnp.exp(0.5 * lv) * eps_ref[...]
        cond = jnp.concatenate([z, gp], axis=1)
        nf = jnp.dot(cond, wdec_ref[...],
                     preferred_element_type=jnp.float32) + bdec_ref[...]
        nf_ref[...] = jnp.broadcast_to(nf, (MAX_LIG_NODES, D))
        mu_ref[...] = mu
        lv_ref[...] = lv
    return pl.pallas_call(
        body,
        out_shape=[jax.ShapeDtypeStruct((MAX_LIG_NODES, D), jnp.float32),
                   jax.ShapeDtypeStruct((1, LATENT), jnp.float32),
                   jax.ShapeDtypeStruct((1, LATENT), jnp.float32)],
    )(po, pd, pb.reshape(1, D), pg.reshape(1, D), pbe.reshape(1, D),
      lo, ld, lb.reshape(1, D), lg.reshape(1, D), lbe.reshape(1, D),
      eps.reshape(1, LATENT), W_mu, b_mu.reshape(1, LATENT),
      W_lv, b_lv.reshape(1, LATENT), W_dec, b_dec.reshape(1, D))


def _fold_edge_selector(p):
    v = p["We"] @ p["a_e"]
    idx = jnp.arange(128)
    vt = jnp.tile(v, 8)
    return jnp.where((idx // 16)[:, None] == jnp.arange(8)[None, :],
                     vt[:, None], 0.0).astype(jnp.float32)


def _attn_vec(p):
    return jnp.stack([p["a_src"], p["a_dst"]], axis=1)


def _pad_edges(ei, E):
    group = NS * NC * BR * CHUNK
    Epad = -(-E // group) * group
    src = jnp.pad(ei[0].astype(jnp.int32), (0, Epad - E))
    dst = jnp.pad(ei[1].astype(jnp.int32), (0, Epad - E))
    return (src.reshape(Epad // CHUNK, CHUNK),
            dst.reshape(Epad // CHUNK, CHUNK), Epad)


def _pad_se(se_flat, Epad):
    E = se_flat.shape[0]
    se = jnp.pad(se_flat, (0, Epad - E), constant_values=NEG_BIG)
    return se.reshape(Epad // CHUNK, CHUNK)


def kernel(prot_x, prot_edge_index, prot_edge_attr,
           lig_x, lig_edge_index, lig_edge_attr, eps, params):
    P = params
    px = jnp.pad(prot_x, ((0, NP - N), (0, 0)))
    lx = jnp.pad(lig_x, ((0, NP - N), (0, 0)))
    psrc, pdst, pEpad = _pad_edges(prot_edge_index, prot_edge_attr.shape[0])
    lsrc, ldst, lEpad = _pad_edges(lig_edge_index, lig_edge_attr.shape[0])
    eap = prot_edge_attr.reshape(-1, 128)
    eal = lig_edge_attr.reshape(-1, 128)

    Bp = jnp.concatenate([_fold_edge_selector(P["p_conv1"]),
                          _fold_edge_selector(P["p_conv2"])], axis=1)
    Bl = jnp.concatenate([_fold_edge_selector(P["l_conv1"]),
                          _fold_edge_selector(P["l_conv2"])], axis=1)

    sep = _edge_scores(eap, Bp)
    sel = _edge_scores(eal, Bl)
    sep1 = _pad_se(sep[:, :8].reshape(-1), pEpad)
    sep2 = _pad_se(sep[:, 8:].reshape(-1), pEpad)
    sel1 = _pad_se(sel[:, :8].reshape(-1), lEpad)
    sel2 = _pad_se(sel[:, 8:].reshape(-1), lEpad)

    nEp = prot_edge_attr.shape[0]
    nEl = lig_edge_attr.shape[0]

    hp1, Sp1 = _dense_node(px, P["p_conv1"]["W"], _attn_vec(P["p_conv1"]))
    hl1, Sl1 = _dense_node(lx, P["l_conv1"]["W"], _attn_vec(P["l_conv1"]))
    op1, dp1 = _edge_pass(psrc, pdst, sep1, Sp1[:, 0], Sp1[:, 1], hp1, nEp)
    ol1, dl1 = _edge_pass(lsrc, ldst, sel1, Sl1[:, 0], Sl1[:, 1], hl1, nEl)
    dp1 = dp1.reshape(NC, NP)
    dl1 = dl1.reshape(NC, NP)

    hp2, Sp2 = _bn_relu_dense(op1, dp1[..., None], P["p_conv1"]["b"],
                              P["p_bn1_g"], P["p_bn1_b"],
                              P["p_conv2"]["W"], _attn_vec(P["p_conv2"]))
    hl2, Sl2 = _bn_relu_dense(ol1, dl1[..., None], P["l_conv1"]["b"],
                              P["l_bn1_g"], P["l_bn1_b"],
                              P["l_conv2"]["W"], _attn_vec(P["l_conv2"]))

    op2, dp2 = _edge_pass(psrc, pdst, sep2, Sp2[:, 0], Sp2[:, 1], hp2, nEp)
    ol2, dl2 = _edge_pass(lsrc, ldst, sel2, Sl2[:, 0], Sl2[:, 1], hl2, nEl)
    dp2 = dp2.reshape(NC, NP)
    dl2 = dl2.reshape(NC, NP)

    nf, mu, lv = _head(op2, dp2[..., None], P["p_conv2"]["b"],
                       P["p_bn2_g"], P["p_bn2_b"],
                       ol2, dl2[..., None], P["l_conv2"]["b"],
                       P["l_bn2_g"], P["l_bn2_b"],
                       eps, P["W_mu"], P["b_mu"], P["W_lv"], P["b_lv"],
                       P["W_dec"], P["b_dec"])
    return (nf[None], mu.reshape(LATENT), lv.reshape(LATENT))
